# Initial kernel scaffold; baseline (speedup 1.0000x reference)
import jax
import jax.numpy as jnp
from jax.experimental import pallas as pl


def kernel(x, edges, edge_index, Wq, Wk, Wv, We_k, We_v, W_ex, W_sq, W_out, b_out, ln_g, ln_b):
    raise NotImplementedError("write your pallas kernel here")


# speedup vs baseline: 6.5940x; 6.5940x over previous
import jax
import jax.numpy as jnp
from jax import lax
from jax.experimental import pallas as pl
from jax.experimental.pallas import tpu as pltpu
from jax.experimental.pallas import tpu_sc as plsc

N = 10000
E = 320000
DIM = 128
HEADS = 8
HEAD_DIM = DIM // HEADS
SCALE = HEAD_DIM ** -0.5

NW = 32
CH = 128
NCHUNKS = E // CH
CHD = 64
NCHUNKS_D = E // CHD


def _ln(x, g, b, eps=1e-5):
    mu = jnp.mean(x, axis=-1, keepdims=True)
    var = jnp.mean((x - mu) ** 2, axis=-1, keepdims=True)
    return (x - mu) / jnp.sqrt(var + eps) * g + b


def _node_proj_body(x_ref, g_ref, b_ref, wq_ref, wk_ref, wv_ref,
                    q_ref, k_ref, v_ref):
    xn = _ln(x_ref[...], g_ref[...], b_ref[...])
    q_ref[...] = jnp.dot(xn, wq_ref[...], preferred_element_type=jnp.float32)
    k_ref[...] = jnp.dot(xn, wk_ref[...], preferred_element_type=jnp.float32) * SCALE
    v_ref[...] = jnp.dot(xn, wv_ref[...], preferred_element_type=jnp.float32)


def _node_proj(x, g, b, Wq, Wk, Wv):
    BN = 1000
    grid = (N // BN,)
    row_spec = pl.BlockSpec((BN, DIM), lambda i: (i, 0))
    full_spec = pl.BlockSpec((DIM, DIM), lambda i: (0, 0))
    vec_spec = pl.BlockSpec((1, DIM), lambda i: (0, 0))
    out = jax.ShapeDtypeStruct((N, DIM), jnp.float32)
    return pl.pallas_call(
        _node_proj_body,
        grid=grid,
        in_specs=[row_spec, vec_spec, vec_spec, full_spec, full_spec, full_spec],
        out_specs=[row_spec, row_spec, row_spec],
        out_shape=[out, out, out],
    )(x, g, b, Wq, Wk, Wv)


def _edge_proj_body(e_ref, g_ref, b_ref, wk_ref, wv_ref, ek_ref, ev_ref):
    en = _ln(e_ref[...], g_ref[...], b_ref[...])
    ek_ref[...] = jnp.dot(en, wk_ref[...], preferred_element_type=jnp.float32) * SCALE
    ev_ref[...] = jnp.dot(en, wv_ref[...], preferred_element_type=jnp.float32)


def _edge_proj(edges, g, b, We_k, We_v):
    BE = 2000
    grid = (E // BE,)
    row_spec = pl.BlockSpec((BE, DIM), lambda i: (i, 0))
    full_spec = pl.BlockSpec((DIM, DIM), lambda i: (0, 0))
    vec_spec = pl.BlockSpec((1, DIM), lambda i: (0, 0))
    out = jax.ShapeDtypeStruct((E, DIM), jnp.float32)
    return pl.pallas_call(
        _edge_proj_body,
        grid=grid,
        in_specs=[row_spec, vec_spec, vec_spec, full_spec, full_spec],
        out_specs=[row_spec, row_spec],
        out_shape=[out, out],
    )(edges, g, b, We_k, We_v)


def _shuffle(v, idx):
    dnums = lax.GatherDimensionNumbers(
        offset_dims=(), collapsed_slice_dims=(0,), start_index_map=(0,))
    return lax.gather(v, idx.reshape(16, 1), dnums, (1,),
                      mode=lax.GatherScatterMode.PROMISE_IN_BOUNDS)


def _sc_logits_body(q_hbm, k_hbm, ek_hbm, dst_hbm, src_hbm, logit_hbm,
                    dsti, srci, qb, kb, ekb, lb, sem0, sem1):
    wid = lax.axis_index("c") * 16 + lax.axis_index("s")
    base_chunks = NCHUNKS // NW
    rem = NCHUNKS - base_chunks * NW
    start = wid * base_chunks + jnp.minimum(wid, rem)
    count = base_chunks + jnp.where(wid < rem, 1, 0)

    lanes = lax.iota(jnp.int32, 16)

    def chunk_body(ci, carry):
        base = ci * CH
        pltpu.sync_copy(dst_hbm.at[pl.ds(base, CH)], dsti)
        pltpu.sync_copy(src_hbm.at[pl.ds(base, CH)], srci)
        cq = pltpu.async_copy(q_hbm.at[dsti], qb, sem0)
        ck = pltpu.async_copy(k_hbm.at[srci], kb, sem1)
        pltpu.sync_copy(ek_hbm.at[pl.ds(base, CH)], ekb)
        cq.wait()
        ck.wait()

        def pair_body(t, c2):
            acc = jnp.zeros((16,), jnp.float32)
            for e in range(2):
                i = t * 2 + e
                for h in range(HEADS):
                    sl = pl.ds(h * HEAD_DIM, HEAD_DIM)
                    pr = qb[i, sl] * (kb[i, sl] + ekb[i, sl])
                    for d in (8, 4, 2, 1):
                        pr = pr + _shuffle(pr, lanes ^ d)
                    acc = jnp.where(lanes == (e * HEADS + h), pr, acc)
            lb[pl.ds(t * 16, 16)] = acc
            return c2

        lax.fori_loop(0, CH // 2, pair_body, 0)
        pltpu.sync_copy(lb, logit_hbm.at[pl.ds(base * HEADS, CH * HEADS)])
        return carry

    lax.fori_loop(start, start + count, chunk_body, 0)


def _sc_logits(q, k, ek, dst, src):
    mesh = plsc.VectorSubcoreMesh(core_axis_name="c", subcore_axis_name="s",
                                  num_cores=2, num_subcores=16)
    kern = pl.kernel(
        _sc_logits_body,
        out_type=jax.ShapeDtypeStruct((E * HEADS,), jnp.float32),
        mesh=mesh,
        scratch_types=[
            pltpu.VMEM((CH,), jnp.int32),
            pltpu.VMEM((CH,), jnp.int32),
            pltpu.VMEM((CH, DIM), jnp.float32),
            pltpu.VMEM((CH, DIM), jnp.float32),
            pltpu.VMEM((CH, DIM), jnp.float32),
            pltpu.VMEM((CH * HEADS,), jnp.float32),
            pltpu.SemaphoreType.DMA,
            pltpu.SemaphoreType.DMA,
        ],
    )
    return kern(q, k, ek, dst, src)


def _attn_mlp_body(l_ref, bdex_ref, bdsq_ref, s_ref):
    t = jnp.dot(l_ref[...], bdex_ref[...], preferred_element_type=jnp.float32)
    t = 0.5 * t * (1.0 + lax.erf(t * (2.0 ** -0.5)))
    t = jnp.dot(t, bdsq_ref[...], preferred_element_type=jnp.float32)
    s_ref[...] = jnp.exp(t)


def _attn_mlp(lflat, BD_ex, BD_sq):
    R = E * HEADS // DIM
    BR = 2000
    grid = (R // BR,)
    row_spec = pl.BlockSpec((BR, DIM), lambda i: (i, 0))
    full_spec = pl.BlockSpec((DIM, DIM), lambda i: (0, 0))
    return pl.pallas_call(
        _attn_mlp_body,
        grid=grid,
        in_specs=[row_spec, full_spec, full_spec],
        out_specs=row_spec,
        out_shape=jax.ShapeDtypeStruct((R, DIM), jnp.float32),
    )(lflat, BD_ex, BD_sq)


SHEADS = 16
ZC = (N + CHD - 1) // CHD
ZCT = ZC // 16
DCH = 16
NDCH = N // DCH
DPT = NDCH // 16


def _sc_accum_body(v_hbm, ev_hbm, s_hbm, dst_hbm, src_hbm,
                   nump_hbm,
                   dsti, srci, gi, vb, evb, sb, num_sp, sem0):
    cid = lax.axis_index("c")
    sid = lax.axis_index("s")
    wid = cid * 16 + sid
    lanes = lax.iota(jnp.int32, 16)

    def vbz(i, c2):
        for j in range(DIM // 16):
            vb[i, pl.ds(j * 16, 16)] = jnp.zeros((16,), jnp.float32)
        return c2

    lax.fori_loop(0, CHD, vbz, 0)

    zrem = ZC - 16 * ZCT
    zstart = sid * ZCT + jnp.minimum(sid, zrem)
    zcount = ZCT + jnp.where(sid < zrem, 1, 0)

    def zbody(zi, c2):
        base = zi * CHD
        for j in range(CHD // 16):
            dsti[pl.ds(j * 16, 16)] = jnp.minimum(lanes + (base + j * 16),
                                                  N - 1)
        pltpu.sync_copy(vb, num_sp.at[dsti])
        return c2

    lax.fori_loop(zstart, zstart + zcount, zbody, 0)
    plsc.subcore_barrier()

    base_chunks = NCHUNKS_D // NW
    rem = NCHUNKS_D - base_chunks * NW
    start = wid * base_chunks + jnp.minimum(wid, rem)
    count = base_chunks + jnp.where(wid < rem, 1, 0)

    def chunk_body(ci, carry):
        base = ci * CHD
        pltpu.sync_copy(dst_hbm.at[pl.ds(base, CHD)], dsti)
        pltpu.sync_copy(src_hbm.at[pl.ds(base, CHD)], srci)
        cv = pltpu.async_copy(v_hbm.at[srci], vb, sem0)
        pltpu.sync_copy(ev_hbm.at[pl.ds(base, CHD)], evb)
        pltpu.sync_copy(s_hbm.at[pl.ds(base, CHD)], sb)
        cv.wait()

        def edge_body(i, c2):
            sv = sb[i, pl.ds(0, SHEADS)]
            for h in range(HEADS):
                sl = pl.ds(h * HEAD_DIM, HEAD_DIM)
                vb[i, sl] = (vb[i, sl] + evb[i, sl]) * sv[h]
            return c2

        lax.fori_loop(0, CHD, edge_body, 0)
        pltpu.sync_copy(vb, num_sp.at[dsti], add=True)
        return carry

    lax.fori_loop(start, start + count, chunk_body, 0)
    plsc.subcore_barrier()

    dstart = sid * DPT
    dcount = DPT + jnp.where(sid == 15, NDCH - 16 * DPT, 0)

    def dbody(zi, c2):
        gi[pl.ds(0, DCH)] = lanes + zi * DCH
        pltpu.async_copy(num_sp.at[gi], vb.at[pl.ds(0, DCH)], sem0).wait()
        pltpu.sync_copy(vb.at[pl.ds(0, DCH)],
                        nump_hbm.at[pl.ds(cid * N + zi * DCH, DCH)])
        return c2

    lax.fori_loop(dstart, dstart + dcount, dbody, 0)


def _sc_accum(v, ev, s, dst, src):
    s16 = jnp.concatenate([s, jnp.zeros((E, HEADS), jnp.float32)], axis=1)
    mesh = plsc.VectorSubcoreMesh(core_axis_name="c", subcore_axis_name="s",
                                  num_cores=2, num_subcores=16)
    kern = pl.kernel(
        _sc_accum_body,
        out_type=jax.ShapeDtypeStruct((2 * N, DIM), jnp.float32),
        mesh=mesh,
        scratch_types=[
            pltpu.VMEM((CHD,), jnp.int32),
            pltpu.VMEM((CHD,), jnp.int32),
            pltpu.VMEM((DCH,), jnp.int32),
            pltpu.VMEM((CHD, DIM), jnp.float32),
            pltpu.VMEM((CHD, DIM), jnp.float32),
            pltpu.VMEM((CHD, SHEADS), jnp.float32),
            pltpu.VMEM_SHARED((N, DIM), jnp.float32),
            pltpu.SemaphoreType.DMA,
        ],
    )
    return kern(v, ev, s16, dst, src)


def _sc_den_body(s_hbm, dst_hbm, denp_hbm,
                 dsti, gi, sb16, sbw, den_sp, sem0):
    cid = lax.axis_index("c")
    sid = lax.axis_index("s")
    wid = cid * 16 + sid
    lanes = lax.iota(jnp.int32, 16)

    def sbz(i, c2):
        for j in range(DIM // 16):
            sbw[i, pl.ds(j * 16, 16)] = jnp.zeros((16,), jnp.float32)
        return c2

    lax.fori_loop(0, CHD, sbz, 0)

    zrem = ZC - 16 * ZCT
    zstart = sid * ZCT + jnp.minimum(sid, zrem)
    zcount = ZCT + jnp.where(sid < zrem, 1, 0)

    def zbody(zi, c2):
        base = zi * CHD
        for j in range(CHD // 16):
            dsti[pl.ds(j * 16, 16)] = jnp.minimum(lanes + (base + j * 16),
                                                  N - 1)
        pltpu.sync_copy(sbw, den_sp.at[dsti])
        return c2

    lax.fori_loop(zstart, zstart + zcount, zbody, 0)
    plsc.subcore_barrier()

    base_chunks = NCHUNKS_D // NW
    rem = NCHUNKS_D - base_chunks * NW
    start = wid * base_chunks + jnp.minimum(wid, rem)
    count = base_chunks + jnp.where(wid < rem, 1, 0)

    def chunk_body(ci, carry):
        base = ci * CHD
        pltpu.sync_copy(dst_hbm.at[pl.ds(base, CHD)], dsti)
        pltpu.sync_copy(s_hbm.at[pl.ds(base, CHD)], sb16)

        def copy_body(i, c2):
            sbw[i, pl.ds(0, SHEADS)] = sb16[i, pl.ds(0, SHEADS)]
            return c2

        lax.fori_loop(0, CHD, copy_body, 0)
        pltpu.sync_copy(sbw, den_sp.at[dsti], add=True)
        return carry

    lax.fori_loop(start, start + count, chunk_body, 0)
    plsc.subcore_barrier()

    dstart = sid * DPT
    dcount = DPT + jnp.where(sid == 15, NDCH - 16 * DPT, 0)

    def dbody(zi, c2):
        gi[pl.ds(0, DCH)] = lanes + zi * DCH
        pltpu.async_copy(den_sp.at[gi], sbw.at[pl.ds(0, DCH)], sem0).wait()
        pltpu.sync_copy(sbw.at[pl.ds(0, DCH)],
                        denp_hbm.at[pl.ds(cid * N + zi * DCH, DCH)])
        return c2

    lax.fori_loop(dstart, dstart + dcount, dbody, 0)


def _sc_den(s16, dst):
    mesh = plsc.VectorSubcoreMesh(core_axis_name="c", subcore_axis_name="s",
                                  num_cores=2, num_subcores=16)
    kern = pl.kernel(
        _sc_den_body,
        out_type=jax.ShapeDtypeStruct((2 * N, DIM), jnp.float32),
        mesh=mesh,
        scratch_types=[
            pltpu.VMEM((CHD,), jnp.int32),
            pltpu.VMEM((DCH,), jnp.int32),
            pltpu.VMEM((CHD, SHEADS), jnp.float32),
            pltpu.VMEM((CHD, DIM), jnp.float32),
            pltpu.VMEM_SHARED((N, DIM), jnp.float32),
            pltpu.SemaphoreType.DMA,
        ],
    )
    return kern(s16, dst)


def _finalize_body(n0_ref, n1_ref, d0_ref, d1_ref, w_ref, b_ref, out_ref):
    num = n0_ref[...] + n1_ref[...]
    den = d0_ref[:, :HEADS] + d1_ref[:, :HEADS] + 1e-16
    bn = num.shape[0]
    den_rep = jnp.broadcast_to(den.reshape(bn, HEADS, 1),
                               (bn, HEADS, HEAD_DIM)).reshape(bn, DIM)
    r = num / den_rep
    out_ref[...] = jnp.dot(r, w_ref[...], preferred_element_type=jnp.float32) + b_ref[...]


def _finalize(n0, n1, d0, d1, WoutT, b):
    BN = 1000
    grid = (N // BN,)
    row_spec = pl.BlockSpec((BN, DIM), lambda i: (i, 0))
    den_spec = pl.BlockSpec((BN, DIM), lambda i: (i, 0))
    full_spec = pl.BlockSpec((DIM, DIM), lambda i: (0, 0))
    vec_spec = pl.BlockSpec((1, DIM), lambda i: (0, 0))
    return pl.pallas_call(
        _finalize_body,
        grid=grid,
        in_specs=[row_spec, row_spec, den_spec, den_spec, full_spec, vec_spec],
        out_specs=row_spec,
        out_shape=jax.ShapeDtypeStruct((N, DIM), jnp.float32),
    )(n0, n1, d0, d1, WoutT, b)


def kernel(x, edges, edge_index, Wq, Wk, Wv, We_k, We_v, W_ex, W_sq,
           W_out, b_out, ln_g, ln_b):
    dst = edge_index[0].astype(jnp.int32)
    src = edge_index[1].astype(jnp.int32)
    g = ln_g.reshape(1, DIM)
    b = ln_b.reshape(1, DIM)

    q, k, v = _node_proj(x, g, b, Wq, Wk, Wv)
    ek, ev = _edge_proj(edges, g, b, We_k, We_v)

    logits = _sc_logits(q, k, ek, dst, src)

    eye16 = jnp.eye(16, dtype=jnp.float32)
    BD_ex = jnp.kron(eye16, W_ex.T)
    BD_sq = jnp.kron(eye16, W_sq.T)
    s = _attn_mlp(logits.reshape(E * HEADS // DIM, DIM), BD_ex, BD_sq)
    s = s.reshape(E, HEADS)

    nump = _sc_accum(v, ev, s, dst, src)
    s16 = jnp.concatenate([s, jnp.zeros((E, HEADS), jnp.float32)], axis=1)
    denp = _sc_den(s16, dst)

    return _finalize(nump[:N], nump[N:], denp[:N], denp[N:],
                     W_out.T, b_out.reshape(1, DIM))
